# Initial kernel scaffold; baseline (speedup 1.0000x reference)
import jax
import jax.numpy as jnp
from jax.experimental import pallas as pl


def kernel(node_feat, edge_index_0, edge_attr_0, edge_index_1, edge_attr_1, params):
    raise NotImplementedError("write your pallas kernel here")


# speedup vs baseline: 2.4628x; 2.4628x over previous
import functools

import jax
import jax.numpy as jnp
from jax import lax
from jax.experimental import pallas as pl
from jax.experimental.pallas import tpu as pltpu
from jax.experimental.pallas import tpu_sc as plsc

N = 50000
D = 128
E = 300000
R = 2
NL = 2

NC, NS, LANES = 2, 16, 16
NW = NC * NS

NCHUNK = 4
CHUNK = 12544
NPAD = NCHUNK * CHUNK
DUMP = CHUNK
RPT = CHUNK // NS

EPW = 9376
EPAD = EPW * NW
BATCH = 128
CAP = 9472

TBLK = 512

f32 = jnp.float32
i32 = jnp.int32


@functools.lru_cache(maxsize=1)
def _sc_kernels():
    mesh = plsc.VectorSubcoreMesh(core_axis_name="c", subcore_axis_name="s",
                                  num_cores=NC, num_subcores=NS)
    cparams = pltpu.CompilerParams(needs_layout_passes=False)
    bin_k = functools.partial(
        pl.kernel,
        out_type=[
            jax.ShapeDtypeStruct((R * NCHUNK * NW * CAP,), i32),
            jax.ShapeDtypeStruct((R * NCHUNK * NW * CAP,), i32),
            jax.ShapeDtypeStruct((R * NCHUNK * NW * CAP,), i32),
            jax.ShapeDtypeStruct((R * NW * NCHUNK * LANES,), i32),
        ],
        mesh=mesh,
        compiler_params=cparams,
        scratch_types=[
            pltpu.VMEM((1184,), i32),
            pltpu.VMEM((1184,), i32),
            pltpu.VMEM((NCHUNK * CAP,), i32),
            pltpu.VMEM((NCHUNK * CAP,), i32),
            pltpu.VMEM((NCHUNK * CAP,), i32),
            pltpu.VMEM((NCHUNK * LANES,), i32),
        ],
    )(_bin_body)
    s_k = functools.partial(
        pl.kernel,
        out_type=jax.ShapeDtypeStruct((NPAD, D), f32),
        mesh=mesh,
        compiler_params=cparams,
        scratch_types=[
            pltpu.VMEM((BATCH,), i32),
            pltpu.VMEM((BATCH,), i32),
            pltpu.VMEM((BATCH, D), f32),
            pltpu.VMEM((NC * NCHUNK * LANES,), i32),
            pltpu.VMEM_SHARED((CHUNK + 1, D), f32),
            pltpu.SemaphoreType.DMA,
        ],
    )(_s_body)
    agg_k = functools.partial(
        pl.kernel,
        out_type=jax.ShapeDtypeStruct((NPAD, D), f32),
        mesh=mesh,
        compiler_params=cparams,
        scratch_types=[
            pltpu.VMEM((BATCH,), i32),
            pltpu.VMEM((BATCH,), i32),
            pltpu.VMEM((BATCH, D), f32),
            pltpu.VMEM((R * NC * NCHUNK * LANES,), i32),
            pltpu.VMEM_SHARED((CHUNK + 1, D), f32),
            pltpu.SemaphoreType.DMA,
        ],
    )(_agg_body)
    return bin_k, s_k, agg_k


def _bin_body(src_hbm, dst_hbm, bin_src, bin_dloc, bin_eid, counts,
              st_s, st_d, seg_s, seg_dl, seg_e, cnt_st):
    cid_ax = lax.axis_index("c")
    sid_ax = lax.axis_index("s")
    wid = sid_ax * NC + cid_ax
    lanes = lax.iota(i32, LANES)
    zeros16 = jnp.zeros((LANES,), i32)
    dump16 = jnp.full((LANES,), DUMP, i32)
    epad16 = jnp.full((LANES,), E, i32)

    for r in range(R):
        def prefill(i, _):
            for c in range(NCHUNK):
                seg_s[pl.ds(c * CAP + i * LANES, LANES)] = zeros16
                seg_dl[pl.ds(c * CAP + i * LANES, LANES)] = dump16
                seg_e[pl.ds(c * CAP + i * LANES, LANES)] = epad16
            return 0
        lax.fori_loop(0, CAP // LANES, prefill, 0)

        ebase = wid * EPW
        offs = (jnp.int32(0),) * NCHUNK
        for blk in range(8):
            blen = 1184 if blk < 7 else 1088
            boff = blk * 1184
            pltpu.sync_copy(
                src_hbm.at[pl.ds(r * EPAD + ebase + boff, blen)],
                st_s.at[pl.ds(0, blen)])
            pltpu.sync_copy(
                dst_hbm.at[pl.ds(r * EPAD + ebase + boff, blen)],
                st_d.at[pl.ds(0, blen)])

            def body(i, offs, boff=boff):
                s = st_s[pl.ds(i * LANES, LANES)]
                d = st_d[pl.ds(i * LANES, LANES)]
                eid = (ebase + boff + i * LANES) + lanes
                cid = ((d >= CHUNK).astype(i32)
                       + (d >= 2 * CHUNK).astype(i32)
                       + (d >= 3 * CHUNK).astype(i32))
                new_offs = []
                for c in range(NCHUNK):
                    oc = offs[c]
                    m = cid == c
                    plsc.store_compressed(
                        seg_s.at[pl.ds(c * CAP + oc, LANES)], s, mask=m)
                    plsc.store_compressed(
                        seg_dl.at[pl.ds(c * CAP + oc, LANES)],
                        d - c * CHUNK, mask=m)
                    plsc.store_compressed(
                        seg_e.at[pl.ds(c * CAP + oc, LANES)], eid, mask=m)
                    new_offs.append(
                        oc + plsc.all_reduce_population_count(m)[0])
                return tuple(new_offs)

            offs = lax.fori_loop(0, blen // LANES, body, offs)

        for c in range(NCHUNK):
            cnt_st[pl.ds(c * LANES, LANES)] = jnp.full((LANES,), offs[c],
                                                       i32)
        pltpu.sync_copy(
            cnt_st,
            counts.at[pl.ds((r * NW + wid) * NCHUNK * LANES,
                            NCHUNK * LANES)])
        for c in range(NCHUNK):
            row = (r * NCHUNK + c) * NW + wid
            pltpu.sync_copy(seg_s.at[pl.ds(c * CAP, CAP)],
                            bin_src.at[pl.ds(row * CAP, CAP)])
            pltpu.sync_copy(seg_dl.at[pl.ds(c * CAP, CAP)],
                            bin_dloc.at[pl.ds(row * CAP, CAP)])
            pltpu.sync_copy(seg_e.at[pl.ds(c * CAP, CAP)],
                            bin_eid.at[pl.ds(row * CAP, CAP)])


def _s_body(a_hbm, bin_eid, bin_dloc, counts, zrows,
            s_out, eidv, dlocv, arows, cnt, acc, sem):
    cid_ax = lax.axis_index("c")
    sid_ax = lax.axis_index("s")
    lanes = lax.iota(i32, LANES)
    pltpu.sync_copy(
        counts.at[pl.ds(2 * sid_ax * NCHUNK * LANES, 2 * NCHUNK * LANES)],
        cnt)

    for cc in range(2):
        chunk = cid_ax + 2 * cc
        pltpu.sync_copy(zrows, acc.at[pl.ds(sid_ax * RPT, RPT)])
        plsc.subcore_barrier()
        for sl in range(2):
            seg = 2 * sid_ax + sl
            n = cnt[pl.ds((sl * NCHUNK + chunk) * LANES, LANES)][0]
            nb = (n + BATCH - 1) // BATCH

            def bbody(b, _, seg=seg):
                row = chunk * NW + seg
                pltpu.sync_copy(
                    bin_eid.at[pl.ds(row * CAP + b * BATCH, BATCH)], eidv)
                pltpu.sync_copy(
                    bin_dloc.at[pl.ds(row * CAP + b * BATCH, BATCH)], dlocv)
                pltpu.async_copy(a_hbm.at[eidv], arows, sem).wait()
                pltpu.sync_copy(arows, acc.at[dlocv], add=True)
                return 0

            lax.fori_loop(0, nb, bbody, 0)
        plsc.subcore_barrier()
        pltpu.sync_copy(
            acc.at[pl.ds(sid_ax * RPT, RPT)],
            s_out.at[pl.ds(chunk * CHUNK + sid_ax * RPT, RPT)])
        plsc.subcore_barrier()


def _agg_body(m0_hbm, m1_hbm, base_hbm, bin_src, bin_dloc, counts,
              agg, srcv, dlocv, rows, cnt, acc, sem):
    cid_ax = lax.axis_index("c")
    sid_ax = lax.axis_index("s")
    lanes = lax.iota(i32, LANES)
    for r in range(R):
        pltpu.sync_copy(
            counts.at[pl.ds((r * NW + 2 * sid_ax) * NCHUNK * LANES,
                            2 * NCHUNK * LANES)],
            cnt.at[pl.ds(r * 2 * NCHUNK * LANES, 2 * NCHUNK * LANES)])

    for cc in range(2):
        chunk = cid_ax + 2 * cc
        rowbase = chunk * CHUNK + sid_ax * RPT
        pltpu.sync_copy(base_hbm.at[pl.ds(rowbase, RPT)],
                        acc.at[pl.ds(sid_ax * RPT, RPT)])
        plsc.subcore_barrier()
        for r in range(R):
            m_hbm = m0_hbm if r == 0 else m1_hbm
            for sl in range(2):
                seg = 2 * sid_ax + sl
                n = cnt[pl.ds(((r * 2 + sl) * NCHUNK + chunk) * LANES,
                              LANES)][0]
                nb = (n + BATCH - 1) // BATCH

                def gbody(b, _, seg=seg, m_hbm=m_hbm, r=r):
                    row = (r * NCHUNK + chunk) * NW + seg
                    pltpu.sync_copy(
                        bin_src.at[pl.ds(row * CAP + b * BATCH, BATCH)],
                        srcv)
                    pltpu.sync_copy(
                        bin_dloc.at[pl.ds(row * CAP + b * BATCH, BATCH)],
                        dlocv)
                    pltpu.async_copy(m_hbm.at[srcv], rows, sem).wait()
                    pltpu.sync_copy(rows, acc.at[dlocv], add=True)
                    return 0

                lax.fori_loop(0, nb, gbody, 0)
        plsc.subcore_barrier()
        pltpu.sync_copy(acc.at[pl.ds(sid_ax * RPT, RPT)],
                        agg.at[pl.ds(rowbase, RPT)])
        plsc.subcore_barrier()


def _ln(x, g, b):
    m = jnp.mean(x, axis=-1, keepdims=True)
    xc = x - m
    v = jnp.mean(xc * xc, axis=-1, keepdims=True)
    return g * xc * lax.rsqrt(v + 1e-5) + b


def _mats_body(h, s, w0, b0, w1, b1, ew, m0_ref, m1_ref, base_ref):
    m0_ref[...] = jnp.dot(h, w0, preferred_element_type=f32) + b0
    m1_ref[...] = jnp.dot(h, w1, preferred_element_type=f32) + b1
    base_ref[...] = jnp.dot(s, ew, preferred_element_type=f32)


def _t_in_body(nf_ref, iw_ref, ib_ref, w0_ref, b0_ref, w1_ref, b1_ref,
               s_ref, ew_ref, h_ref, m0_ref, m1_ref, base_ref):
    h = jnp.dot(nf_ref[...], iw_ref[...], preferred_element_type=f32) \
        + ib_ref[...]
    h_ref[...] = h
    _mats_body(h, s_ref[...], w0_ref[...], b0_ref[...], w1_ref[...],
               b1_ref[...], ew_ref[...], m0_ref, m1_ref, base_ref)


def _t_mid_body(hp_ref, ag_ref, g_ref, be_ref, w0_ref, b0_ref, w1_ref,
                b1_ref, s_ref, ew_ref, h_ref, m0_ref, m1_ref, base_ref):
    x = hp_ref[...] + jnp.maximum(ag_ref[...], 0.0)
    h = _ln(x, g_ref[...], be_ref[...])
    h_ref[...] = h
    _mats_body(h, s_ref[...], w0_ref[...], b0_ref[...], w1_ref[...],
               b1_ref[...], ew_ref[...], m0_ref, m1_ref, base_ref)


def _t_out_body(hp_ref, ag_ref, g_ref, be_ref, h_ref):
    x = hp_ref[...] + jnp.maximum(ag_ref[...], 0.0)
    h_ref[...] = _ln(x, g_ref[...], be_ref[...])


_row_spec = pl.BlockSpec((TBLK, D), lambda i: (i, 0))
_w_spec = pl.BlockSpec((D, D), lambda i: (0, 0))
_b_spec = pl.BlockSpec((1, D), lambda i: (0, 0))
_GRID = (NPAD // TBLK,)
_sds = jax.ShapeDtypeStruct((NPAD, D), f32)

_t_in = pl.pallas_call(
    _t_in_body, grid=_GRID,
    in_specs=[_row_spec, _w_spec, _b_spec, _w_spec, _b_spec, _w_spec,
              _b_spec, _row_spec, _w_spec],
    out_specs=[_row_spec] * 4, out_shape=[_sds] * 4)

_t_mid = pl.pallas_call(
    _t_mid_body, grid=_GRID,
    in_specs=[_row_spec, _row_spec, _b_spec, _b_spec, _w_spec, _b_spec,
              _w_spec, _b_spec, _row_spec, _w_spec],
    out_specs=[_row_spec] * 4, out_shape=[_sds] * 4)

_t_out = pl.pallas_call(
    _t_out_body, grid=_GRID,
    in_specs=[_row_spec, _row_spec, _b_spec, _b_spec],
    out_specs=_row_spec, out_shape=_sds)


def kernel(node_feat, edge_index_0, edge_attr_0, edge_index_1, edge_attr_1,
           params):
    del edge_attr_1
    nf = jnp.concatenate(
        [node_feat[0], jnp.zeros((NPAD - N, D), f32)], axis=0)
    pad_src = jnp.zeros((EPAD - E,), i32)
    pad_dst = jnp.full((EPAD - E,), NPAD - 1, i32)
    src_all = jnp.concatenate([
        edge_index_0[0], pad_src, edge_index_1[0], pad_src])
    dst_all = jnp.concatenate([
        edge_index_0[1], pad_dst, edge_index_1[1], pad_dst])
    a_rows = jnp.concatenate(
        [edge_attr_0, jnp.ones((E, 1), f32), jnp.zeros((E, D - 5), f32)],
        axis=1)
    a_rows = jnp.concatenate([a_rows, jnp.zeros((EPAD - E, D), f32)],
                             axis=0)
    zrows = jnp.zeros((RPT, D), f32)

    bin_k, s_k, agg_k = _sc_kernels()
    bin_src, bin_dloc, bin_eid, counts = bin_k(src_all, dst_all)
    s_mat = s_k(a_rows, bin_eid, bin_dloc, counts, zrows)

    layers = params["layers"]

    def ew_mat(layer):
        return jnp.concatenate(
            [layer["edge_W"][0], layer["edge_b"][0].reshape(1, D),
             jnp.zeros((D - 5, D), f32)], axis=0)

    l0 = layers[0]
    h, m0, m1, base = _t_in(
        nf, params["input_W"], params["input_b"].reshape(1, D),
        l0["node_W"][0], l0["node_b"][0].reshape(1, D),
        l0["node_W"][1], l0["node_b"][1].reshape(1, D),
        s_mat, ew_mat(l0))

    for li in range(NL):
        agg = agg_k(m0, m1, base, bin_src, bin_dloc, counts)
        lg = layers[li]["gamma"].reshape(1, D)
        lb = layers[li]["beta"].reshape(1, D)
        if li < NL - 1:
            nxt = layers[li + 1]
            h, m0, m1, base = _t_mid(
                h, agg, lg, lb,
                nxt["node_W"][0], nxt["node_b"][0].reshape(1, D),
                nxt["node_W"][1], nxt["node_b"][1].reshape(1, D),
                s_mat, ew_mat(nxt))
        else:
            h = _t_out(h, agg, lg, lb)

    return h[:N].reshape(1, N, D)
